# baseline (speedup 1.0000x reference)
import jax
import jax.numpy as jnp
from jax import lax
from jax.experimental import pallas as pl
from jax.experimental.pallas import tpu as pltpu
from jax.experimental.pallas import tpu_sc as plsc

N = 1024
E = 65536
T = 15
G1 = 8
H1 = 1200
C = 6
TP = 16
NTILES = 32
EPW = E // NTILES
GPW = EPW // 16
NPW = N // NTILES
NB = 16
NCHUNK = 8
NPC = N // NCHUNK
MSTEPS = NPC // NB


def _mesh():
    return plsc.VectorSubcoreMesh(core_axis_name="c", subcore_axis_name="s")


def _wid():
    return lax.axis_index("s") * 2 + lax.axis_index("c")


def _deg_body(dst_hbm, degp_hbm, dst_v, deg_v):
    w = _wid()
    pltpu.sync_copy(dst_hbm.at[w], dst_v)

    def zero(n, carry):
        deg_v[pl.ds(n * 16, 16)] = jnp.zeros((16,), jnp.float32)
        return carry

    lax.fori_loop(0, N, zero, 0)

    def body(g, carry):
        d16 = dst_v[pl.ds(g * 16, 16)]
        for j in range(16):
            d = d16[j]
            deg_v[pl.ds(d * 16, 16)] = deg_v[pl.ds(d * 16, 16)] + 1.0
        return carry

    lax.fori_loop(0, GPW, body, 0)
    pltpu.sync_copy(deg_v, degp_hbm.at[w])


def _deg_call(dst2):
    return pl.kernel(
        _deg_body,
        out_type=jax.ShapeDtypeStruct((NTILES, N * 16), jnp.float32),
        scratch_types=[
            pltpu.VMEM((EPW,), jnp.int32),
            pltpu.VMEM((N * 16,), jnp.float32),
        ],
        mesh=_mesh(),
    )(dst2)


def _dinv_body(degp_ref, out_ref):
    deg = jnp.sum(degp_ref[...], axis=0)
    out_ref[...] = lax.rsqrt(deg + 1.0)


def _dinv_call(degp):
    return pl.pallas_call(
        _dinv_body,
        out_shape=jax.ShapeDtypeStruct((N, 16), jnp.float32),
    )(degp)


def _prop_body(src_hbm, dst_hbm, x_hbm, dinv_hbm, aggp_hbm,
               src_v, dst_v, x_v, dinv_v, acc_v):
    w = _wid()
    pltpu.sync_copy(src_hbm.at[w], src_v)
    pltpu.sync_copy(dst_hbm.at[w], dst_v)
    pltpu.sync_copy(x_hbm, x_v)
    pltpu.sync_copy(dinv_hbm, dinv_v)

    def zero(n, carry):
        acc_v[pl.ds(n * 16, 16)] = jnp.zeros((16,), jnp.float32)
        return carry

    lax.fori_loop(0, N, zero, 0)

    def body(g, carry):
        s16 = src_v[pl.ds(g * 16, 16)]
        d16 = dst_v[pl.ds(g * 16, 16)]
        for j in range(16):
            s = s16[j]
            d = d16[j]
            nrm = (dinv_v[pl.ds(s * 16, 16)] * dinv_v[pl.ds(d * 16, 16)])
            acc_v[pl.ds(d * 16, 16)] = (acc_v[pl.ds(d * 16, 16)]
                                        + x_v[pl.ds(s * 16, 16)] * nrm)
        return carry

    lax.fori_loop(0, GPW, body, 0)

    base = w * NPW
    def selfloop(r, carry):
        n = base + r
        dv = dinv_v[pl.ds(n * 16, 16)]
        acc_v[pl.ds(n * 16, 16)] = (acc_v[pl.ds(n * 16, 16)]
                                    + x_v[pl.ds(n * 16, 16)] * (dv * dv))
        return carry

    lax.fori_loop(0, NPW, selfloop, 0)
    pltpu.sync_copy(acc_v, aggp_hbm.at[w])


def _prop_call(src2, dst2, xflat, dinv):
    return pl.kernel(
        _prop_body,
        out_type=jax.ShapeDtypeStruct((NTILES, N * TP), jnp.float32),
        scratch_types=[
            pltpu.VMEM((EPW,), jnp.int32),
            pltpu.VMEM((EPW,), jnp.int32),
            pltpu.VMEM((N * TP,), jnp.float32),
            pltpu.VMEM((N * 16,), jnp.float32),
            pltpu.VMEM((N * TP,), jnp.float32),
        ],
        mesh=_mesh(),
    )(src2, dst2, xflat, dinv)


def _flat_body(aggp_ref, w_ref, b_ref, out_ref):
    aggx = jnp.sum(aggp_ref[...], axis=0)
    col = lax.broadcasted_iota(jnp.int32, (TP, T * G1), 1)
    row = lax.broadcasted_iota(jnp.int32, (TP, T * G1), 0)
    sel_t = (col // G1 == row).astype(jnp.float32)
    a2 = jnp.dot(aggx, sel_t, preferred_element_type=jnp.float32)
    colg = lax.broadcasted_iota(jnp.int32, (G1, T * G1), 1)
    rowg = lax.broadcasted_iota(jnp.int32, (G1, T * G1), 0)
    sel_g = (colg % G1 == rowg).astype(jnp.float32)
    wv = jnp.dot(w_ref[...], sel_g, preferred_element_type=jnp.float32)
    bv = jnp.dot(b_ref[...], sel_g, preferred_element_type=jnp.float32)
    out_ref[...] = jnp.maximum(a2 * wv + bv, 0.0)


def _flat_call(aggp, conv_W, conv_b):
    return pl.pallas_call(
        _flat_body,
        out_shape=jax.ShapeDtypeStruct((N, T * G1), jnp.float32),
    )(aggp, conv_W, conv_b)


def _mv_body(flat_ref, w1_ref, z1_ref, acc_ref):
    k = pl.program_id(0)

    @pl.when(k == 0)
    def _init():
        acc_ref[...] = jnp.zeros_like(acc_ref)

    fblk = flat_ref[...]
    acc = acc_ref[...]
    for r in range(NB):
        acc = acc + jnp.dot(fblk[r:r + 1, :], w1_ref[r],
                            preferred_element_type=jnp.float32)
    acc_ref[...] = acc

    @pl.when(k == pl.num_programs(0) - 1)
    def _final():
        z1_ref[...] = acc_ref[...]


def _mv_call(flatc, w13):
    return pl.pallas_call(
        _mv_body,
        grid=(MSTEPS,),
        in_specs=[
            pl.BlockSpec((NB, T * G1), lambda k: (k, 0)),
            pl.BlockSpec((NB, T * G1, H1), lambda k: (k, 0, 0)),
        ],
        out_specs=pl.BlockSpec((1, H1), lambda k: (0, 0)),
        out_shape=jax.ShapeDtypeStruct((1, H1), jnp.float32),
        scratch_shapes=[pltpu.VMEM((1, H1), jnp.float32)],
    )(flatc, w13)


def _head_body(z1p_ref, b1_ref, w2_ref, b2_ref, out_ref):
    z1 = jnp.sum(z1p_ref[...], axis=0, keepdims=True) + b1_ref[...]
    z2 = jnp.dot(z1, w2_ref[...], preferred_element_type=jnp.float32)
    z2 = z2 + b2_ref[...]
    m = jnp.max(z2, axis=1, keepdims=True)
    lse = jnp.log(jnp.sum(jnp.exp(z2 - m), axis=1, keepdims=True)) + m
    out_ref[...] = z2 - lse


def _head_call(z1p, fc1_b, fc2_W, fc2_b):
    return pl.pallas_call(
        _head_body,
        out_shape=jax.ShapeDtypeStruct((1, C), jnp.float32),
    )(z1p, fc1_b, fc2_W, fc2_b)


def _impl(x, graph_list, conv_W, conv_b, fc1_W, fc1_b, fc2_W, fc2_b):
    src2 = graph_list[0, 0, 0].reshape(NTILES, EPW)
    dst2 = graph_list[0, 0, 1].reshape(NTILES, EPW)
    xflat = jnp.pad(x[0], ((0, 0), (0, TP - T))).reshape(N * TP)
    degp = _deg_call(dst2)
    dinv = _dinv_call(degp.reshape(NTILES, N, 16)).reshape(N * 16)
    aggp = _prop_call(src2, dst2, xflat, dinv)
    flat = _flat_call(aggp.reshape(NTILES, N, TP), conv_W,
                      conv_b.reshape(1, G1))
    rpc = NPC * T * G1
    parts = []
    for k in range(NCHUNK):
        w13 = fc1_W[k * rpc:(k + 1) * rpc].reshape(NPC, T * G1, H1)
        parts.append(_mv_call(flat[k * NPC:(k + 1) * NPC], w13))
    z1p = jnp.concatenate(parts, axis=0)
    return _head_call(z1p, fc1_b.reshape(1, H1), fc2_W, fc2_b.reshape(1, C))


_pipeline = jax.jit(_impl)


def kernel(x, graph_list, edge_weight_list, mapping_list, conv_W, conv_b,
           fc1_W, fc1_b, fc2_W, fc2_b):
    return _pipeline(x, graph_list, conv_W, conv_b, fc1_W, fc1_b,
                     fc2_W, fc2_b)


# speedup vs baseline: 1.7803x; 1.7803x over previous
import jax
import jax.numpy as jnp
from jax import lax
from jax.experimental import pallas as pl
from jax.experimental.pallas import tpu as pltpu
from jax.experimental.pallas import tpu_sc as plsc

N = 1024
E = 65536
T = 15
G1 = 8
H1 = 1200
C = 6
TP = 16
NTILES = 32
EPW = E // NTILES
GPW = EPW // 16
NPW = N // NTILES
NB = 16
MSTEPS = N // NB


def _mesh():
    return plsc.VectorSubcoreMesh(core_axis_name="c", subcore_axis_name="s")


def _wid():
    return lax.axis_index("s") * 2 + lax.axis_index("c")


def _deg_body(dst_hbm, degp_hbm, dst_v, deg_v):
    w = _wid()
    pltpu.sync_copy(dst_hbm.at[w], dst_v)

    def zero(n, carry):
        deg_v[pl.ds(n * 16, 16)] = jnp.zeros((16,), jnp.float32)
        return carry

    lax.fori_loop(0, N, zero, 0)

    def body(g, carry):
        d16 = dst_v[pl.ds(g * 16, 16)]
        for j in range(16):
            d = d16[j]
            deg_v[pl.ds(d * 16, 16)] = deg_v[pl.ds(d * 16, 16)] + 1.0
        return carry

    lax.fori_loop(0, GPW, body, 0)
    pltpu.sync_copy(deg_v, degp_hbm.at[w])


def _deg_call(dst2):
    return pl.kernel(
        _deg_body,
        out_type=jax.ShapeDtypeStruct((NTILES, N * 16), jnp.float32),
        scratch_types=[
            pltpu.VMEM((EPW,), jnp.int32),
            pltpu.VMEM((N * 16,), jnp.float32),
        ],
        mesh=_mesh(),
    )(dst2)


def _dinv_body(degp_ref, out_ref):
    deg = jnp.sum(degp_ref[...], axis=0)
    out_ref[...] = lax.rsqrt(deg + 1.0)


def _dinv_call(degp):
    return pl.pallas_call(
        _dinv_body,
        out_shape=jax.ShapeDtypeStruct((N, 16), jnp.float32),
    )(degp)


def _prop_body(src_hbm, dst_hbm, x_hbm, dinv_hbm, aggp_hbm,
               src_v, dst_v, x_v, dinv_v, acc_v):
    w = _wid()
    pltpu.sync_copy(src_hbm.at[w], src_v)
    pltpu.sync_copy(dst_hbm.at[w], dst_v)
    pltpu.sync_copy(x_hbm, x_v)
    pltpu.sync_copy(dinv_hbm, dinv_v)

    def zero(n, carry):
        acc_v[pl.ds(n * 16, 16)] = jnp.zeros((16,), jnp.float32)
        return carry

    lax.fori_loop(0, N, zero, 0)

    def body(g, carry):
        s16 = src_v[pl.ds(g * 16, 16)]
        d16 = dst_v[pl.ds(g * 16, 16)]
        for j in range(16):
            s = s16[j]
            d = d16[j]
            nrm = (dinv_v[pl.ds(s * 16, 16)] * dinv_v[pl.ds(d * 16, 16)])
            acc_v[pl.ds(d * 16, 16)] = (acc_v[pl.ds(d * 16, 16)]
                                        + x_v[pl.ds(s * 16, 16)] * nrm)
        return carry

    lax.fori_loop(0, GPW, body, 0)

    base = w * NPW
    def selfloop(r, carry):
        n = base + r
        dv = dinv_v[pl.ds(n * 16, 16)]
        acc_v[pl.ds(n * 16, 16)] = (acc_v[pl.ds(n * 16, 16)]
                                    + x_v[pl.ds(n * 16, 16)] * (dv * dv))
        return carry

    lax.fori_loop(0, NPW, selfloop, 0)
    pltpu.sync_copy(acc_v, aggp_hbm.at[w])


def _prop_call(src2, dst2, xflat, dinv):
    return pl.kernel(
        _prop_body,
        out_type=jax.ShapeDtypeStruct((NTILES, N * TP), jnp.float32),
        scratch_types=[
            pltpu.VMEM((EPW,), jnp.int32),
            pltpu.VMEM((EPW,), jnp.int32),
            pltpu.VMEM((N * TP,), jnp.float32),
            pltpu.VMEM((N * 16,), jnp.float32),
            pltpu.VMEM((N * TP,), jnp.float32),
        ],
        mesh=_mesh(),
    )(src2, dst2, xflat, dinv)


def _mv_body(aggp_ref, cw_ref, cb_ref, w1_ref, b1_ref, w2_ref, b2_ref,
             out_ref, flat_s, z1_ref):
    k = pl.program_id(0)

    @pl.when(k == 0)
    def _init():
        aggx = jnp.sum(aggp_ref[...], axis=0)
        col = lax.broadcasted_iota(jnp.int32, (TP, T * G1), 1)
        row = lax.broadcasted_iota(jnp.int32, (TP, T * G1), 0)
        sel_t = (col // G1 == row).astype(jnp.float32)
        a2 = jnp.dot(aggx, sel_t, preferred_element_type=jnp.float32)
        colg = lax.broadcasted_iota(jnp.int32, (G1, T * G1), 1)
        rowg = lax.broadcasted_iota(jnp.int32, (G1, T * G1), 0)
        sel_g = (colg % G1 == rowg).astype(jnp.float32)
        wv = jnp.dot(cw_ref[...], sel_g, preferred_element_type=jnp.float32)
        bv = jnp.dot(cb_ref[...], sel_g, preferred_element_type=jnp.float32)
        flat_s[...] = jnp.maximum(a2 * wv + bv, 0.0)
        z1_ref[...] = jnp.zeros_like(z1_ref)
        out_ref[...] = jnp.zeros_like(out_ref)

    fblk = flat_s[pl.ds(k * NB, NB), :]
    acc = z1_ref[...]
    for r in range(NB):
        acc = acc + jnp.dot(fblk[r:r + 1, :].astype(jnp.bfloat16), w1_ref[r],
                            preferred_element_type=jnp.float32)
    z1_ref[...] = acc

    @pl.when(k == pl.num_programs(0) - 1)
    def _final():
        z1 = z1_ref[...] + b1_ref[...]
        z2 = jnp.dot(z1, w2_ref[...], preferred_element_type=jnp.float32)
        z2 = z2 + b2_ref[...]
        m = jnp.max(z2, axis=1, keepdims=True)
        lse = jnp.log(jnp.sum(jnp.exp(z2 - m), axis=1, keepdims=True)) + m
        out_ref[...] = z2 - lse


def _mv_call(aggp, conv_W, conv_b, w13, fc1_b, fc2_W, fc2_b):
    return pl.pallas_call(
        _mv_body,
        grid=(MSTEPS,),
        in_specs=[
            pl.BlockSpec((NTILES, N, TP), lambda k: (0, 0, 0)),
            pl.BlockSpec((1, G1), lambda k: (0, 0)),
            pl.BlockSpec((1, G1), lambda k: (0, 0)),
            pl.BlockSpec((NB, T * G1, H1), lambda k: (k, 0, 0)),
            pl.BlockSpec((1, H1), lambda k: (0, 0)),
            pl.BlockSpec((H1, C), lambda k: (0, 0)),
            pl.BlockSpec((1, C), lambda k: (0, 0)),
        ],
        out_specs=pl.BlockSpec((1, C), lambda k: (0, 0)),
        out_shape=jax.ShapeDtypeStruct((1, C), jnp.float32),
        scratch_shapes=[
            pltpu.VMEM((N, T * G1), jnp.float32),
            pltpu.VMEM((1, H1), jnp.float32),
        ],
    )(aggp, conv_W, conv_b, w13, fc1_b, fc2_W, fc2_b)


def _impl(x, graph_list, conv_W, conv_b, fc1_W, fc1_b, fc2_W, fc2_b):
    src2 = graph_list[0, 0, 0].reshape(NTILES, EPW)
    dst2 = graph_list[0, 0, 1].reshape(NTILES, EPW)
    xflat = jnp.pad(x[0], ((0, 0), (0, TP - T))).reshape(N * TP)
    degp = _deg_call(dst2)
    dinv = _dinv_call(degp.reshape(NTILES, N, 16)).reshape(N * 16)
    aggp = _prop_call(src2, dst2, xflat, dinv)
    w13 = fc1_W.reshape(N, T * G1, H1).astype(jnp.bfloat16)
    return _mv_call(aggp.reshape(NTILES, N, TP), conv_W,
                    conv_b.reshape(1, G1), w13, fc1_b.reshape(1, H1),
                    fc2_W, fc2_b.reshape(1, C))


_pipeline = jax.jit(_impl)


def kernel(x, graph_list, edge_weight_list, mapping_list, conv_W, conv_b,
           fc1_W, fc1_b, fc2_W, fc2_b):
    return _pipeline(x, graph_list, conv_W, conv_b, fc1_W, fc1_b,
                     fc2_W, fc2_b)
